# baseline (speedup 1.0000x reference)
import functools

import jax
import jax.numpy as jnp
from jax import lax
from jax.experimental import pallas as pl
from jax.experimental.pallas import tpu as pltpu
from jax.experimental.pallas import tpu_sc as plsc


def _embed(idx_t, tab_pad, pos_flat, *, b, s, d, nc, ns):
    n_workers = nc * ns
    bw = b // n_workers
    mesh = plsc.VectorSubcoreMesh(core_axis_name="c", subcore_axis_name="s")

    @functools.partial(
        pl.kernel,
        out_type=jax.ShapeDtypeStruct((s, d, b), jnp.float32),
        mesh=mesh,
        scratch_types=[
            [pltpu.VMEM((bw,), jnp.int32) for _ in range(4)],
            [pltpu.VMEM((bw, 128), jnp.float32) for _ in range(4)],
            [pltpu.VMEM((d, bw), jnp.float32) for _ in range(2)],
            pltpu.VMEM((s * d,), jnp.float32),
            [pltpu.SemaphoreType.DMA for _ in range(4)],
            [pltpu.SemaphoreType.DMA for _ in range(4)],
            [pltpu.SemaphoreType.DMA for _ in range(2)],
        ],
        compiler_params=pltpu.CompilerParams(use_tc_tiling_on_sc=True,
                                             needs_layout_passes=False),
    )
    def run(idx_hbm, tab_hbm, pos_hbm, out_hbm, idx, rows, tr, pos_v, si, sg,
            so):
        wid = lax.axis_index("s") * nc + lax.axis_index("c")
        b0 = wid * bw
        pltpu.sync_copy(pos_hbm, pos_v)
        dvecs = [lax.iota(jnp.int32, 16) + 16 * c for c in range(d // 16)]
        for p in range(3):
            pltpu.sync_copy(idx_hbm.at[p, pl.ds(b0, bw)], idx[p])
            pltpu.async_copy(tab_hbm.at[idx[p]], rows[p], sg[p])
        pltpu.async_copy(idx_hbm.at[3, pl.ds(b0, bw)], idx[3], si[3])

        def unit(i, ph):
            q = (ph + 3) % 4
            t = ph % 2
            pltpu.make_async_copy(tab_hbm.at[idx[ph]], rows[ph],
                                  sg[ph]).wait()

            @pl.when(i + 4 < s)
            def _():
                pltpu.async_copy(idx_hbm.at[i + 4, pl.ds(b0, bw)], idx[ph],
                                 si[ph])

            @pl.when(i + 3 < s)
            def _():
                pltpu.make_async_copy(idx_hbm.at[0, pl.ds(0, bw)], idx[q],
                                      si[q]).wait()
                pltpu.async_copy(tab_hbm.at[idx[q]], rows[q], sg[q])

            @pl.when(i >= 2)
            def _():
                pltpu.make_async_copy(tr[t], out_hbm.at[0, :, pl.ds(0, bw)],
                                      so[t]).wait()

            pos_cs = [pos_v[pl.ds(i * d + 16 * c, 16)]
                      for c in range(d // 16)]

            def row_body(r, carry):
                rvec = jnp.full((16,), r, jnp.int32)
                for c in range(d // 16):
                    x = rows[ph][r, pl.ds(16 * c, 16)] + pos_cs[c]
                    plsc.store_scatter(tr[t], [dvecs[c], rvec], x)
                return carry

            lax.fori_loop(0, bw, row_body, 0, unroll=8)
            pltpu.async_copy(tr[t], out_hbm.at[i, :, pl.ds(b0, bw)], so[t])

        def quad_body(j, carry):
            for p in range(4):
                unit(4 * j + p, p)
            return carry

        lax.fori_loop(0, s // 4, quad_body, 0)
        pltpu.make_async_copy(tr[0], out_hbm.at[0, :, pl.ds(0, bw)],
                              so[0]).wait()
        pltpu.make_async_copy(tr[1], out_hbm.at[0, :, pl.ds(0, bw)],
                              so[1]).wait()

    return run(idx_t, tab_pad, pos_flat)


def kernel(input_ids, token_table, position_table):
    b, s = input_ids.shape
    v, d = token_table.shape
    info = plsc.get_sparse_core_info()
    nc, ns = info.num_cores, info.num_subcores

    idx_t = input_ids.T
    pos_flat = position_table[:s].reshape(-1)
    tab_pad = jnp.pad(token_table, ((0, 0), (0, 128 - d)))
    out_t = _embed(idx_t, tab_pad, pos_flat, b=b, s=s, d=d, nc=nc, ns=ns)
    return out_t.transpose(2, 0, 1)


# speedup vs baseline: 1.0532x; 1.0532x over previous
import functools

import jax
import jax.numpy as jnp
from jax import lax
from jax.experimental import pallas as pl
from jax.experimental.pallas import tpu as pltpu
from jax.experimental.pallas import tpu_sc as plsc


def _embed(idx_flat, token_table, pos_ext, *, n_rows, d, n_workers, chunk,
           seq_len, nc):
    per_w = n_rows // n_workers
    n_chunks = per_w // chunk
    assert n_chunks % 2 == 0
    mesh = plsc.VectorSubcoreMesh(core_axis_name="c", subcore_axis_name="s")

    @functools.partial(
        pl.kernel,
        out_type=jax.ShapeDtypeStruct((n_rows, d), jnp.float32),
        mesh=mesh,
        scratch_types=[
            pltpu.VMEM((chunk,), jnp.int32),
            pltpu.VMEM((chunk,), jnp.int32),
            pltpu.VMEM((chunk, d), jnp.float32),
            pltpu.VMEM((chunk, d), jnp.float32),
            pltpu.VMEM(pos_ext.shape, jnp.float32),
            pltpu.SemaphoreType.DMA,
            pltpu.SemaphoreType.DMA,
            pltpu.SemaphoreType.DMA,
            pltpu.SemaphoreType.DMA,
            pltpu.SemaphoreType.DMA,
            pltpu.SemaphoreType.DMA,
        ],
        compiler_params=pltpu.CompilerParams(use_tc_tiling_on_sc=False),
    )
    def run(idx_hbm, tab_hbm, pos_hbm, out_hbm, idx_a, idx_b, rows_a, rows_b,
            pos_v, sg_a, sg_b, si_a, si_b, so_a, so_b):
        wid = lax.axis_index("s") * nc + lax.axis_index("c")
        base_w = wid * per_w
        pltpu.sync_copy(pos_hbm, pos_v)
        pltpu.sync_copy(idx_hbm.at[pl.ds(base_w, chunk)], idx_a)
        pltpu.async_copy(tab_hbm.at[idx_a], rows_a, sg_a)
        pltpu.async_copy(idx_hbm.at[pl.ds(base_w + chunk, chunk)],
                         idx_b, si_b)

        def half(i, cur, idx_c, idx_n, rows_c, rows_n, sg_c, sg_n, si_c, si_n,
                 so_n):
            pltpu.make_async_copy(tab_hbm.at[idx_c], rows_c, sg_c).wait()

            @pl.when(i + 1 < n_chunks)
            def _():
                @pl.when(i >= 1)
                def _():
                    pltpu.make_async_copy(
                        rows_n, out_hbm.at[pl.ds(0, chunk)], so_n).wait()

                pltpu.make_async_copy(idx_hbm.at[pl.ds(0, chunk)],
                                      idx_n, si_n).wait()
                pltpu.async_copy(tab_hbm.at[idx_n], rows_n, sg_n)

            @pl.when(i + 2 < n_chunks)
            def _():
                pltpu.async_copy(
                    idx_hbm.at[pl.ds(base_w + (i + 2) * chunk, chunk)],
                    idx_c, si_c)

            f0 = base_w + i * chunk
            m = lax.rem(f0, seq_len)

            def add_body(r, c2):
                pr = m + r
                for cc in range(d // 16):
                    plsc.addupdate(
                        rows_c.at[r, pl.ds(cc * 16, 16)],
                        pos_v[pr, pl.ds(cc * 16, 16)],
                    )
                return c2

            lax.fori_loop(0, chunk, add_body, 0, unroll=8)
            pltpu.async_copy(rows_c, out_hbm.at[pl.ds(f0, chunk)],
                             so_a if cur == 0 else so_b)

        def pair_body(j, carry):
            half(2 * j, 0, idx_a, idx_b, rows_a, rows_b, sg_a, sg_b, si_a,
                 si_b, so_b)
            half(2 * j + 1, 1, idx_b, idx_a, rows_b, rows_a, sg_b, sg_a,
                 si_b, si_a, so_a)
            return carry

        lax.fori_loop(0, n_chunks // 2, pair_body, 0)
        pltpu.make_async_copy(rows_a, out_hbm.at[pl.ds(0, chunk)],
                              so_a).wait()
        pltpu.make_async_copy(rows_b, out_hbm.at[pl.ds(0, chunk)],
                              so_b).wait()

    return run(idx_flat, token_table, pos_ext)


def kernel(input_ids, token_table, position_table):
    b, s = input_ids.shape
    v, d = token_table.shape
    n_rows = b * s
    chunk = 512
    info = plsc.get_sparse_core_info()
    nc, ns = info.num_cores, info.num_subcores
    n_workers = nc * ns

    reps = -(-(s + chunk) // s)
    pos_ext = jnp.concatenate([position_table[:s]] * reps, axis=0)[: s + chunk]

    idx_flat = input_ids.reshape(-1)
    out = _embed(idx_flat, token_table, pos_ext, n_rows=n_rows, d=d,
                 n_workers=n_workers, chunk=chunk, seq_len=s, nc=nc)
    return out.reshape(b, s, d)


# speedup vs baseline: 1.3181x; 1.2516x over previous
import functools

import jax
import jax.numpy as jnp
from jax import lax
from jax.experimental import pallas as pl
from jax.experimental.pallas import tpu as pltpu
from jax.experimental.pallas import tpu_sc as plsc


def _embed(idx_flat, token_table, pos_ext, *, n_rows, d, n_workers, chunk,
           seq_len, nc):
    per_w = n_rows // n_workers
    n_chunks = per_w // chunk
    assert n_chunks % 2 == 0
    mesh = plsc.VectorSubcoreMesh(core_axis_name="c", subcore_axis_name="s")

    @functools.partial(
        pl.kernel,
        out_type=jax.ShapeDtypeStruct((n_rows, 128), jnp.float32),
        mesh=mesh,
        scratch_types=[
            pltpu.VMEM((chunk,), jnp.int32),
            pltpu.VMEM((chunk,), jnp.int32),
            pltpu.VMEM((chunk, d), jnp.float32),
            pltpu.VMEM((chunk, d), jnp.float32),
            pltpu.VMEM(pos_ext.shape, jnp.float32),
            pltpu.SemaphoreType.DMA,
            pltpu.SemaphoreType.DMA,
            pltpu.SemaphoreType.DMA,
            pltpu.SemaphoreType.DMA,
            pltpu.SemaphoreType.DMA,
            pltpu.SemaphoreType.DMA,
        ],
        compiler_params=pltpu.CompilerParams(use_tc_tiling_on_sc=False),
    )
    def run(idx_hbm, tab_hbm, pos_hbm, out_hbm, idx_a, idx_b, rows_a, rows_b,
            pos_v, sg_a, sg_b, si_a, si_b, so_a, so_b):
        wid = lax.axis_index("s") * nc + lax.axis_index("c")
        base_w = wid * per_w
        pltpu.sync_copy(pos_hbm, pos_v)
        pltpu.sync_copy(idx_hbm.at[pl.ds(base_w, chunk)], idx_a)
        pltpu.async_copy(tab_hbm.at[idx_a], rows_a, sg_a)
        pltpu.async_copy(idx_hbm.at[pl.ds(base_w + chunk, chunk)],
                         idx_b, si_b)

        def half(i, cur, idx_c, idx_n, rows_c, rows_n, sg_c, sg_n, si_c, si_n,
                 so_n):
            pltpu.make_async_copy(tab_hbm.at[idx_c], rows_c, sg_c).wait()

            @pl.when(i + 1 < n_chunks)
            def _():
                @pl.when(i >= 1)
                def _():
                    pltpu.make_async_copy(
                        rows_n,
                        out_hbm.at[pl.ds(0, chunk), pl.ds(0, d)],
                        so_n).wait()

                pltpu.make_async_copy(idx_hbm.at[pl.ds(0, chunk)],
                                      idx_n, si_n).wait()
                pltpu.async_copy(tab_hbm.at[idx_n], rows_n, sg_n)

            @pl.when(i + 2 < n_chunks)
            def _():
                pltpu.async_copy(
                    idx_hbm.at[pl.ds(base_w + (i + 2) * chunk, chunk)],
                    idx_c, si_c)

            f0 = base_w + i * chunk
            m = lax.rem(f0, seq_len)

            def add_body(r, c2):
                pr = m + r
                for cc in range(d // 16):
                    plsc.addupdate(
                        rows_c.at[r, pl.ds(cc * 16, 16)],
                        pos_v[pr, pl.ds(cc * 16, 16)],
                    )
                return c2

            lax.fori_loop(0, chunk, add_body, 0, unroll=8)
            pltpu.async_copy(rows_c,
                             out_hbm.at[pl.ds(f0, chunk), pl.ds(0, d)],
                             so_a if cur == 0 else so_b)

        def pair_body(j, carry):
            half(2 * j, 0, idx_a, idx_b, rows_a, rows_b, sg_a, sg_b, si_a,
                 si_b, so_b)
            half(2 * j + 1, 1, idx_b, idx_a, rows_b, rows_a, sg_b, sg_a,
                 si_b, si_a, so_a)
            return carry

        lax.fori_loop(0, n_chunks // 2, pair_body, 0)
        pltpu.make_async_copy(rows_a, out_hbm.at[pl.ds(0, chunk), pl.ds(0, d)],
                              so_a).wait()
        pltpu.make_async_copy(rows_b, out_hbm.at[pl.ds(0, chunk), pl.ds(0, d)],
                              so_b).wait()

    return run(idx_flat, token_table, pos_ext)


def kernel(input_ids, token_table, position_table):
    b, s = input_ids.shape
    v, d = token_table.shape
    n_rows = b * s
    chunk = 512
    info = plsc.get_sparse_core_info()
    nc, ns = info.num_cores, info.num_subcores
    n_workers = nc * ns

    reps = -(-(s + chunk) // s)
    pos_ext = jnp.concatenate([position_table[:s]] * reps, axis=0)[: s + chunk]

    idx_flat = input_ids.reshape(-1)
    out = _embed(idx_flat, token_table, pos_ext, n_rows=n_rows, d=d,
                 n_workers=n_workers, chunk=chunk, seq_len=s, nc=nc)
    return out[:, :d].reshape(b, s, d)
